# baseline (speedup 1.0000x reference)
import functools

import jax
import jax.numpy as jnp
from jax.experimental import pallas as pl
from jax.experimental.pallas import tpu as pltpu

E = 8
K = 2
D = 768
F = 3072
FB = 512
TM = 128
S = 2048
N = S * K
G = N // TM + E
NPAD = G * TM


def _router_body(x_ref, cw_ref, cb_ref, w_ref, sel_ref, rw_ref, pay_ref):
    x = x_ref[...]
    cw = cw_ref[...]
    logits = jax.lax.dot_general(
        x, cw, (((1,), (1,)), ((), ())), preferred_element_type=jnp.float32)
    logits = logits + cb_ref[...]
    bids = (1.0 / (1.0 + jnp.exp(-logits))) * w_ref[...]
    s = bids.shape[0]
    neg_inf = jnp.float32(float('-inf'))
    ei = jax.lax.broadcasted_iota(jnp.int32, (s, E), 1)
    max1 = jnp.max(bids, axis=1, keepdims=True)
    idx1 = jnp.min(jnp.where(bids == max1, ei, E), axis=1, keepdims=True)
    b2 = jnp.where(ei == idx1, neg_inf, bids)
    max2 = jnp.max(b2, axis=1, keepdims=True)
    idx2 = jnp.min(jnp.where(b2 == max2, ei, E), axis=1, keepdims=True)
    b3 = jnp.where(ei == idx2, neg_inf, b2)
    max3 = jnp.max(b3, axis=1, keepdims=True)
    t = jnp.exp(max2 - max1)
    rw1 = 1.0 / (1.0 + t)
    rw2 = t / (1.0 + t)
    sel_ref[...] = jnp.concatenate([idx1, idx2], axis=1)
    rw_ref[...] = jnp.concatenate([rw1, rw2], axis=1)
    pay_ref[...] = jnp.concatenate([max3, max3], axis=1)


def _ffn_body(eid_ref, used_ref, xs_ref, gw_ref, uw_ref, dw_ref, ys_ref,
              gws_ref, uws_ref, dws_ref):
    f = pl.program_id(0)
    g = pl.program_id(1)

    prev_e = eid_ref[jnp.maximum(g, 1) - 1]
    new_block = (g == 0) | (eid_ref[g] != prev_e)

    @pl.when(new_block)
    def _cast():
        gws_ref[...] = gw_ref[0].astype(jnp.bfloat16)
        uws_ref[...] = uw_ref[0].astype(jnp.bfloat16)
        dws_ref[...] = dw_ref[0].astype(jnp.bfloat16)

    @pl.when(g < used_ref[0])
    def _compute():
        xb = xs_ref[pl.ds(g * TM, TM), :].astype(jnp.bfloat16)
        gm = jax.lax.dot(xb, gws_ref[...], preferred_element_type=jnp.float32)
        um = jax.lax.dot(xb, uws_ref[...], preferred_element_type=jnp.float32)
        h = (gm / (1.0 + jnp.exp(-gm))) * um
        contrib = jax.lax.dot(h.astype(jnp.bfloat16), dws_ref[...],
                              preferred_element_type=jnp.float32)

        @pl.when(f == 0)
        def _set():
            ys_ref[pl.ds(g * TM, TM), :] = contrib

        @pl.when(f != 0)
        def _acc():
            ys_ref[pl.ds(g * TM, TM), :] += contrib


def _grouped_ffn(xs, gate_w, up_w, down_w, eid, used):
    grid_spec = pltpu.PrefetchScalarGridSpec(
        num_scalar_prefetch=2,
        grid=(F // FB, G),
        in_specs=[
            pl.BlockSpec((NPAD, D), lambda f, g, eid, used: (0, 0)),
            pl.BlockSpec((1, D, FB), lambda f, g, eid, used: (eid[g], 0, f)),
            pl.BlockSpec((1, D, FB), lambda f, g, eid, used: (eid[g], 0, f)),
            pl.BlockSpec((1, FB, D), lambda f, g, eid, used: (eid[g], f, 0)),
        ],
        out_specs=pl.BlockSpec((NPAD, D), lambda f, g, eid, used: (0, 0)),
        scratch_shapes=[
            pltpu.VMEM((D, FB), jnp.bfloat16),
            pltpu.VMEM((D, FB), jnp.bfloat16),
            pltpu.VMEM((FB, D), jnp.bfloat16),
        ],
    )
    return pl.pallas_call(
        _ffn_body,
        grid_spec=grid_spec,
        out_shape=jax.ShapeDtypeStruct((NPAD, D), jnp.float32),
    )(eid, used, xs, gate_w, up_w, down_w)


def kernel(hidden_states, conf_w, conf_b, gate_w, up_w, down_w, wealth):
    b, s, d = hidden_states.shape
    x = hidden_states.reshape(s, d)

    sel, rw, pay = pl.pallas_call(
        _router_body,
        out_shape=(
            jax.ShapeDtypeStruct((s, K), jnp.int32),
            jax.ShapeDtypeStruct((s, K), jnp.float32),
            jax.ShapeDtypeStruct((s, K), jnp.float32),
        ),
    )(x, conf_w, conf_b.reshape(1, E), wealth.reshape(1, E))

    flat_e = jnp.concatenate([sel[:, 0], sel[:, 1]])
    onehot = jax.nn.one_hot(flat_e, E, dtype=jnp.int32)
    counts = jnp.sum(onehot, axis=0)
    ptiles = (counts + TM - 1) // TM
    used = jnp.sum(ptiles, keepdims=True).astype(jnp.int32)
    poff = jnp.concatenate([jnp.zeros((1,), jnp.int32),
                            jnp.cumsum(ptiles * TM)]).astype(jnp.int32)
    goff = jnp.concatenate([jnp.zeros((1,), jnp.int32),
                            jnp.cumsum(counts)]).astype(jnp.int32)
    order = jnp.argsort(flat_e, stable=True)
    e_sorted = flat_e[order]
    rank = jnp.arange(N, dtype=jnp.int32) - goff[e_sorted]
    pos_sorted = poff[e_sorted] + rank
    pos = jnp.zeros((N,), jnp.int32).at[order].set(pos_sorted)
    tile_iota = jnp.arange(G, dtype=jnp.int32)
    eid = (jnp.sum((tile_iota[:, None] >= (poff[1:] // TM)[None, :]),
                   axis=1)).astype(jnp.int32)
    eid = jnp.minimum(eid, E - 1)
    xs = jnp.zeros((NPAD, D), jnp.float32)
    xs = xs.at[pos[:S]].set(x)
    xs = xs.at[pos[S:]].set(x)

    ys = _grouped_ffn(xs, gate_w, up_w, down_w, eid, used)

    out = rw[:, 0:1] * ys[pos[:S]] + rw[:, 1:2] * ys[pos[S:]]

    return (out.reshape(b, s, d), sel.reshape(b, s, K),
            rw.reshape(b, s, K), pay.reshape(b, s, K))


# speedup vs baseline: 1.2053x; 1.2053x over previous
import functools

import jax
import jax.numpy as jnp
from jax import lax
from jax.experimental import pallas as pl
from jax.experimental.pallas import tpu as pltpu
from jax.experimental.pallas import tpu_sc as plsc

E = 8
K = 2
D = 768
F = 3072
FB = 512
TM = 128
S = 2048
N = S * K
G = N // TM + E
NPAD = G * TM
NW = 32
CHUNK = S // NW
SB = 512


def _router_body(x_ref, cw_ref, cb_ref, w_ref,
                 sel_ref, rw_ref, pay_ref, pos_ref, eid_ref, used_ref):
    x = x_ref[...]
    cw = cw_ref[...]
    logits = jax.lax.dot_general(
        x, cw, (((1,), (1,)), ((), ())), preferred_element_type=jnp.float32)
    logits = logits + cb_ref[...]
    bids = (1.0 / (1.0 + jnp.exp(-logits))) * w_ref[...]
    neg_inf = jnp.float32(float('-inf'))
    ei = jax.lax.broadcasted_iota(jnp.int32, (S, E), 1)
    max1 = jnp.max(bids, axis=1, keepdims=True)
    idx1 = jnp.min(jnp.where(bids == max1, ei, E), axis=1, keepdims=True)
    b2 = jnp.where(ei == idx1, neg_inf, bids)
    max2 = jnp.max(b2, axis=1, keepdims=True)
    idx2 = jnp.min(jnp.where(b2 == max2, ei, E), axis=1, keepdims=True)
    b3 = jnp.where(ei == idx2, neg_inf, b2)
    max3 = jnp.max(b3, axis=1, keepdims=True)
    t = jnp.exp(max2 - max1)
    rw1 = 1.0 / (1.0 + t)
    rw2 = t / (1.0 + t)
    sel_ref[...] = jnp.concatenate([idx1, idx2], axis=1)
    rw_ref[...] = jnp.concatenate([rw1, rw2], axis=1)
    pay_ref[...] = jnp.concatenate([max3, max3], axis=1)

    oh0 = (ei == idx1).astype(jnp.float32)
    oh1 = (ei == idx2).astype(jnp.float32)
    cnt0 = jnp.sum(oh0, axis=0, keepdims=True)
    cnt = cnt0 + jnp.sum(oh1, axis=0, keepdims=True)
    pc = jnp.floor((cnt + (TM - 1)) * (1.0 / TM)) * TM
    ue = jax.lax.broadcasted_iota(jnp.int32, (E, E), 0)
    ve = jax.lax.broadcasted_iota(jnp.int32, (E, E), 1)
    upper = (ue < ve).astype(jnp.float32)
    off = jax.lax.dot(pc, upper, preferred_element_type=jnp.float32)
    incl = off + pc

    ri = jax.lax.broadcasted_iota(jnp.int32, (SB, SB), 0)
    ci = jax.lax.broadcasted_iota(jnp.int32, (SB, SB), 1)
    tri = (ri > ci).astype(jnp.float32)
    prev0 = jnp.zeros((1, E), jnp.float32)
    prev1 = jnp.zeros((1, E), jnp.float32)
    for blk in range(S // SB):
        ohb0 = oh0[blk * SB:(blk + 1) * SB, :]
        ohb1 = oh1[blk * SB:(blk + 1) * SB, :]
        r0 = jax.lax.dot(tri, ohb0, preferred_element_type=jnp.float32) + prev0
        r1 = jax.lax.dot(tri, ohb1, preferred_element_type=jnp.float32) + prev1
        rank0 = jnp.sum(r0 * ohb0, axis=1, keepdims=True)
        rank1 = jnp.sum(r1 * ohb1, axis=1, keepdims=True)
        base0 = jnp.sum(ohb0 * off, axis=1, keepdims=True)
        base1 = jnp.sum(ohb1 * (off + cnt0), axis=1, keepdims=True)
        p0 = (base0 + rank0).astype(jnp.int32)
        p1 = (base1 + rank1).astype(jnp.int32)
        pos_ref[pl.ds(blk * SB, SB), :] = jnp.concatenate([p0, p1], axis=1)
        prev0 = prev0 + jnp.sum(ohb0, axis=0, keepdims=True)
        prev1 = prev1 + jnp.sum(ohb1, axis=0, keepdims=True)

    gi = jax.lax.broadcasted_iota(jnp.int32, (E, G), 1).astype(jnp.float32) * TM
    th = jnp.broadcast_to(incl.reshape(E, 1), (E, G))
    acc = jnp.sum((gi >= th).astype(jnp.int32), axis=0, keepdims=True)
    eid_ref[...] = jnp.minimum(acc, E - 1)
    used_ref[...] = (jnp.sum(pc, axis=1, keepdims=True) *
                     (1.0 / TM)).astype(jnp.int32)


def _scatter_body(x_hbm, p01_hbm, xs_hbm, p0_v, p1_v, rows_v, sem):
    wid = lax.axis_index("s") * 2 + lax.axis_index("c")
    base = wid * CHUNK
    pltpu.sync_copy(p01_hbm.at[0, pl.ds(base, CHUNK)], p0_v)
    pltpu.sync_copy(p01_hbm.at[1, pl.ds(base, CHUNK)], p1_v)
    pltpu.sync_copy(x_hbm.at[pl.ds(base, CHUNK)], rows_v)
    pltpu.async_copy(rows_v, xs_hbm.at[p0_v], sem).wait()
    pltpu.async_copy(rows_v, xs_hbm.at[p1_v], sem).wait()


def _gather_body(ys_hbm, p01_hbm, y0_hbm, y1_hbm, p0_v, p1_v, r0_v, r1_v, sem):
    wid = lax.axis_index("s") * 2 + lax.axis_index("c")
    base = wid * CHUNK
    pltpu.sync_copy(p01_hbm.at[0, pl.ds(base, CHUNK)], p0_v)
    pltpu.sync_copy(p01_hbm.at[1, pl.ds(base, CHUNK)], p1_v)
    pltpu.async_copy(ys_hbm.at[p0_v], r0_v, sem).wait()
    pltpu.async_copy(ys_hbm.at[p1_v], r1_v, sem).wait()
    pltpu.sync_copy(r0_v, y0_hbm.at[pl.ds(base, CHUNK)])
    pltpu.sync_copy(r1_v, y1_hbm.at[pl.ds(base, CHUNK)])


def _combine_body(y0_ref, y1_ref, rw_ref, out_ref):
    w0 = rw_ref[:, 0:1]
    w1 = rw_ref[:, 1:2]
    out_ref[...] = w0 * y0_ref[...] + w1 * y1_ref[...]


def _ffn_body(eid_ref, used_ref, xs_ref, gw_ref, uw_ref, dw_ref, ys_ref,
              gws_ref, uws_ref, dws_ref):
    f = pl.program_id(0)
    g = pl.program_id(1)

    prev_e = eid_ref[jnp.maximum(g, 1) - 1]
    new_block = (g == 0) | (eid_ref[g] != prev_e)

    @pl.when(new_block)
    def _cast():
        gws_ref[...] = gw_ref[0].astype(jnp.bfloat16)
        uws_ref[...] = uw_ref[0].astype(jnp.bfloat16)
        dws_ref[...] = dw_ref[0].astype(jnp.bfloat16)

    @pl.when(g < used_ref[0])
    def _compute():
        xb = xs_ref[pl.ds(g * TM, TM), :].astype(jnp.bfloat16)
        gm = jax.lax.dot(xb, gws_ref[...], preferred_element_type=jnp.float32)
        um = jax.lax.dot(xb, uws_ref[...], preferred_element_type=jnp.float32)
        h = (gm / (1.0 + jnp.exp(-gm))) * um
        contrib = jax.lax.dot(h.astype(jnp.bfloat16), dws_ref[...],
                              preferred_element_type=jnp.float32)

        @pl.when(f == 0)
        def _set():
            ys_ref[pl.ds(g * TM, TM), :] = contrib

        @pl.when(f != 0)
        def _acc():
            ys_ref[pl.ds(g * TM, TM), :] += contrib


def _grouped_ffn(xs, gate_w, up_w, down_w, eid, used):
    grid_spec = pltpu.PrefetchScalarGridSpec(
        num_scalar_prefetch=2,
        grid=(F // FB, G),
        in_specs=[
            pl.BlockSpec((NPAD, D), lambda f, g, eid, used: (0, 0)),
            pl.BlockSpec((1, D, FB), lambda f, g, eid, used: (eid[g], 0, f)),
            pl.BlockSpec((1, D, FB), lambda f, g, eid, used: (eid[g], 0, f)),
            pl.BlockSpec((1, FB, D), lambda f, g, eid, used: (eid[g], f, 0)),
        ],
        out_specs=pl.BlockSpec((NPAD, D), lambda f, g, eid, used: (0, 0)),
        scratch_shapes=[
            pltpu.VMEM((D, FB), jnp.bfloat16),
            pltpu.VMEM((D, FB), jnp.bfloat16),
            pltpu.VMEM((FB, D), jnp.bfloat16),
        ],
    )
    return pl.pallas_call(
        _ffn_body,
        grid_spec=grid_spec,
        out_shape=jax.ShapeDtypeStruct((NPAD, D), jnp.float32),
    )(eid, used, xs, gate_w, up_w, down_w)


@functools.cache
def _sc_mesh():
    return plsc.VectorSubcoreMesh(core_axis_name="c", subcore_axis_name="s")


def _scatter(x, p01):
    kfn = functools.partial(
        pl.kernel, mesh=_sc_mesh(),
        out_type=jax.ShapeDtypeStruct((NPAD, D), jnp.float32),
        scratch_types=[
            pltpu.VMEM((CHUNK,), jnp.int32),
            pltpu.VMEM((CHUNK,), jnp.int32),
            pltpu.VMEM((CHUNK, D), jnp.float32),
            pltpu.SemaphoreType.DMA,
        ],
    )(_scatter_body)
    return kfn(x, p01)


def _gather(ys, p01):
    kfn = functools.partial(
        pl.kernel, mesh=_sc_mesh(),
        out_type=(
            jax.ShapeDtypeStruct((S, D), jnp.float32),
            jax.ShapeDtypeStruct((S, D), jnp.float32),
        ),
        scratch_types=[
            pltpu.VMEM((CHUNK,), jnp.int32),
            pltpu.VMEM((CHUNK,), jnp.int32),
            pltpu.VMEM((CHUNK, D), jnp.float32),
            pltpu.VMEM((CHUNK, D), jnp.float32),
            pltpu.SemaphoreType.DMA,
        ],
    )(_gather_body)
    return kfn(ys, p01)


def kernel(hidden_states, conf_w, conf_b, gate_w, up_w, down_w, wealth):
    b, s, d = hidden_states.shape
    x = hidden_states.reshape(s, d)

    sel, rw, pay, pos, eid2d, used2d = pl.pallas_call(
        _router_body,
        out_shape=(
            jax.ShapeDtypeStruct((S, K), jnp.int32),
            jax.ShapeDtypeStruct((S, K), jnp.float32),
            jax.ShapeDtypeStruct((S, K), jnp.float32),
            jax.ShapeDtypeStruct((S, 2), jnp.int32),
            jax.ShapeDtypeStruct((1, G), jnp.int32),
            jax.ShapeDtypeStruct((1, 1), jnp.int32),
        ),
    )(x, conf_w, conf_b.reshape(1, E), wealth.reshape(1, E))

    p01 = pos.T
    xs = _scatter(x, p01)
    ys = _grouped_ffn(xs, gate_w, up_w, down_w,
                      eid2d.reshape(G), used2d.reshape(1))
    y0, y1 = _gather(ys, p01)

    out = pl.pallas_call(
        _combine_body,
        out_shape=jax.ShapeDtypeStruct((S, D), jnp.float32),
    )(y0, y1, rw)

    return (out.reshape(b, s, d), sel.reshape(b, s, K),
            rw.reshape(b, s, K), pay.reshape(b, s, K))


# speedup vs baseline: 1.4136x; 1.1728x over previous
import functools

import jax
import jax.numpy as jnp
from jax import lax
from jax.experimental import pallas as pl
from jax.experimental.pallas import tpu as pltpu
from jax.experimental.pallas import tpu_sc as plsc

E = 8
K = 2
D = 768
F = 3072
FB = 512
TM = 256
S = 2048
N = S * K
G = N // TM + E
NPAD = G * TM
NW = 32
CHUNK = S // NW
SB = 512


def _router_body(x_ref, cw_ref, cb_ref, w_ref,
                 sel_ref, rw_ref, pay_ref, pos_ref, eid_ref, used_ref):
    x = x_ref[...]
    cw = cw_ref[...]
    logits = jax.lax.dot_general(
        x, cw, (((1,), (1,)), ((), ())), preferred_element_type=jnp.float32)
    logits = logits + cb_ref[...]
    bids = (1.0 / (1.0 + jnp.exp(-logits))) * w_ref[...]
    neg_inf = jnp.float32(float('-inf'))
    ei = jax.lax.broadcasted_iota(jnp.int32, (S, E), 1)
    max1 = jnp.max(bids, axis=1, keepdims=True)
    idx1 = jnp.min(jnp.where(bids == max1, ei, E), axis=1, keepdims=True)
    b2 = jnp.where(ei == idx1, neg_inf, bids)
    max2 = jnp.max(b2, axis=1, keepdims=True)
    idx2 = jnp.min(jnp.where(b2 == max2, ei, E), axis=1, keepdims=True)
    b3 = jnp.where(ei == idx2, neg_inf, b2)
    max3 = jnp.max(b3, axis=1, keepdims=True)
    t = jnp.exp(max2 - max1)
    rw1 = 1.0 / (1.0 + t)
    rw2 = t / (1.0 + t)
    sel_ref[...] = jnp.concatenate([idx1, idx2], axis=1)
    rw_ref[...] = jnp.concatenate([rw1, rw2], axis=1)
    pay_ref[...] = jnp.concatenate([max3, max3], axis=1)

    oh0 = (ei == idx1).astype(jnp.float32)
    oh1 = (ei == idx2).astype(jnp.float32)
    cnt0 = jnp.sum(oh0, axis=0, keepdims=True)
    cnt = cnt0 + jnp.sum(oh1, axis=0, keepdims=True)
    pc = jnp.floor((cnt + (TM - 1)) * (1.0 / TM)) * TM
    ue = jax.lax.broadcasted_iota(jnp.int32, (E, E), 0)
    ve = jax.lax.broadcasted_iota(jnp.int32, (E, E), 1)
    upper = (ue < ve).astype(jnp.float32)
    off = jax.lax.dot(pc, upper, preferred_element_type=jnp.float32)
    incl = off + pc

    ri = jax.lax.broadcasted_iota(jnp.int32, (SB, SB), 0)
    ci = jax.lax.broadcasted_iota(jnp.int32, (SB, SB), 1)
    tri = (ri > ci).astype(jnp.float32)
    prev0 = jnp.zeros((1, E), jnp.float32)
    prev1 = jnp.zeros((1, E), jnp.float32)
    for blk in range(S // SB):
        ohb0 = oh0[blk * SB:(blk + 1) * SB, :]
        ohb1 = oh1[blk * SB:(blk + 1) * SB, :]
        r0 = jax.lax.dot(tri, ohb0, preferred_element_type=jnp.float32) + prev0
        r1 = jax.lax.dot(tri, ohb1, preferred_element_type=jnp.float32) + prev1
        rank0 = jnp.sum(r0 * ohb0, axis=1, keepdims=True)
        rank1 = jnp.sum(r1 * ohb1, axis=1, keepdims=True)
        base0 = jnp.sum(ohb0 * off, axis=1, keepdims=True)
        base1 = jnp.sum(ohb1 * (off + cnt0), axis=1, keepdims=True)
        p0 = (base0 + rank0).astype(jnp.int32)
        p1 = (base1 + rank1).astype(jnp.int32)
        pos_ref[pl.ds(blk * SB, SB), :] = jnp.concatenate([p0, p1], axis=1)
        prev0 = prev0 + jnp.sum(ohb0, axis=0, keepdims=True)
        prev1 = prev1 + jnp.sum(ohb1, axis=0, keepdims=True)

    gi = jax.lax.broadcasted_iota(jnp.int32, (E, G), 1).astype(jnp.float32) * TM
    th = jnp.broadcast_to(incl.reshape(E, 1), (E, G))
    acc = jnp.sum((gi >= th).astype(jnp.int32), axis=0, keepdims=True)
    eid_ref[...] = jnp.minimum(acc, E - 1)
    used_ref[...] = (jnp.sum(pc, axis=1, keepdims=True) *
                     (1.0 / TM)).astype(jnp.int32)


def _scatter_body(x_hbm, p01_hbm, xs_hbm, p0_v, p1_v, rows_v, sem):
    wid = lax.axis_index("s") * 2 + lax.axis_index("c")
    base = wid * CHUNK
    pltpu.sync_copy(p01_hbm.at[0, pl.ds(base, CHUNK)], p0_v)
    pltpu.sync_copy(p01_hbm.at[1, pl.ds(base, CHUNK)], p1_v)
    pltpu.sync_copy(x_hbm.at[pl.ds(base, CHUNK)], rows_v)
    pltpu.async_copy(rows_v, xs_hbm.at[p0_v], sem).wait()
    pltpu.async_copy(rows_v, xs_hbm.at[p1_v], sem).wait()


def _gather_body(ys_hbm, p01_hbm, y0_hbm, y1_hbm, p0_v, p1_v, r0_v, r1_v, sem):
    wid = lax.axis_index("s") * 2 + lax.axis_index("c")
    base = wid * CHUNK
    pltpu.sync_copy(p01_hbm.at[0, pl.ds(base, CHUNK)], p0_v)
    pltpu.sync_copy(p01_hbm.at[1, pl.ds(base, CHUNK)], p1_v)
    pltpu.async_copy(ys_hbm.at[p0_v], r0_v, sem).wait()
    pltpu.async_copy(ys_hbm.at[p1_v], r1_v, sem).wait()
    pltpu.sync_copy(r0_v, y0_hbm.at[pl.ds(base, CHUNK)])
    pltpu.sync_copy(r1_v, y1_hbm.at[pl.ds(base, CHUNK)])


def _combine_body(y0_ref, y1_ref, rw_ref, out_ref):
    w0 = rw_ref[:, 0:1]
    w1 = rw_ref[:, 1:2]
    out_ref[...] = w0 * y0_ref[...] + w1 * y1_ref[...]


def _ffn_body(eid_ref, used_ref, xs_ref, gw_ref, uw_ref, dw_ref, ys_ref,
              gws_ref, uws_ref, dws_ref):
    f = pl.program_id(0)
    g = pl.program_id(1)

    prev_e = eid_ref[jnp.maximum(g, 1) - 1]
    new_block = (g == 0) | (eid_ref[g] != prev_e)

    @pl.when(new_block)
    def _cast():
        gws_ref[...] = gw_ref[0].astype(jnp.bfloat16)
        uws_ref[...] = uw_ref[0].astype(jnp.bfloat16)
        dws_ref[...] = dw_ref[0].astype(jnp.bfloat16)

    @pl.when(g < used_ref[0])
    def _compute():
        xb = xs_ref[pl.ds(g * TM, TM), :].astype(jnp.bfloat16)
        gm = jax.lax.dot(xb, gws_ref[...], preferred_element_type=jnp.float32)
        um = jax.lax.dot(xb, uws_ref[...], preferred_element_type=jnp.float32)
        h = (gm / (1.0 + jnp.exp(-gm))) * um
        contrib = jax.lax.dot(h.astype(jnp.bfloat16), dws_ref[...],
                              preferred_element_type=jnp.float32)

        @pl.when(f == 0)
        def _set():
            ys_ref[pl.ds(g * TM, TM), :] = contrib

        @pl.when(f != 0)
        def _acc():
            ys_ref[pl.ds(g * TM, TM), :] += contrib


def _grouped_ffn(xs, gate_w, up_w, down_w, eid, used):
    grid_spec = pltpu.PrefetchScalarGridSpec(
        num_scalar_prefetch=2,
        grid=(F // FB, G),
        in_specs=[
            pl.BlockSpec((NPAD, D), lambda f, g, eid, used: (0, 0)),
            pl.BlockSpec((1, D, FB), lambda f, g, eid, used: (eid[g], 0, f)),
            pl.BlockSpec((1, D, FB), lambda f, g, eid, used: (eid[g], 0, f)),
            pl.BlockSpec((1, FB, D), lambda f, g, eid, used: (eid[g], f, 0)),
        ],
        out_specs=pl.BlockSpec((NPAD, D), lambda f, g, eid, used: (0, 0)),
        scratch_shapes=[
            pltpu.VMEM((D, FB), jnp.bfloat16),
            pltpu.VMEM((D, FB), jnp.bfloat16),
            pltpu.VMEM((FB, D), jnp.bfloat16),
        ],
    )
    return pl.pallas_call(
        _ffn_body,
        grid_spec=grid_spec,
        out_shape=jax.ShapeDtypeStruct((NPAD, D), jnp.float32),
    )(eid, used, xs, gate_w, up_w, down_w)


@functools.cache
def _sc_mesh():
    return plsc.VectorSubcoreMesh(core_axis_name="c", subcore_axis_name="s")


def _scatter(x, p01):
    kfn = functools.partial(
        pl.kernel, mesh=_sc_mesh(),
        out_type=jax.ShapeDtypeStruct((NPAD, D), jnp.float32),
        scratch_types=[
            pltpu.VMEM((CHUNK,), jnp.int32),
            pltpu.VMEM((CHUNK,), jnp.int32),
            pltpu.VMEM((CHUNK, D), jnp.float32),
            pltpu.SemaphoreType.DMA,
        ],
    )(_scatter_body)
    return kfn(x, p01)


def _gather(ys, p01):
    kfn = functools.partial(
        pl.kernel, mesh=_sc_mesh(),
        out_type=(
            jax.ShapeDtypeStruct((S, D), jnp.float32),
            jax.ShapeDtypeStruct((S, D), jnp.float32),
        ),
        scratch_types=[
            pltpu.VMEM((CHUNK,), jnp.int32),
            pltpu.VMEM((CHUNK,), jnp.int32),
            pltpu.VMEM((CHUNK, D), jnp.float32),
            pltpu.VMEM((CHUNK, D), jnp.float32),
            pltpu.SemaphoreType.DMA,
        ],
    )(_gather_body)
    return kfn(ys, p01)


def kernel(hidden_states, conf_w, conf_b, gate_w, up_w, down_w, wealth):
    b, s, d = hidden_states.shape
    x = hidden_states.reshape(s, d)

    sel, rw, pay, pos, eid2d, used2d = pl.pallas_call(
        _router_body,
        out_shape=(
            jax.ShapeDtypeStruct((S, K), jnp.int32),
            jax.ShapeDtypeStruct((S, K), jnp.float32),
            jax.ShapeDtypeStruct((S, K), jnp.float32),
            jax.ShapeDtypeStruct((S, 2), jnp.int32),
            jax.ShapeDtypeStruct((1, G), jnp.int32),
            jax.ShapeDtypeStruct((1, 1), jnp.int32),
        ),
    )(x, conf_w, conf_b.reshape(1, E), wealth.reshape(1, E))

    p01 = pos.T
    xs = _scatter(x, p01)
    ys = _grouped_ffn(xs, gate_w, up_w, down_w,
                      eid2d.reshape(G), used2d.reshape(1))
    y0, y1 = _gather(ys, p01)

    out = pl.pallas_call(
        _combine_body,
        out_shape=jax.ShapeDtypeStruct((S, D), jnp.float32),
    )(y0, y1, rw)

    return (out.reshape(b, s, d), sel.reshape(b, s, K),
            rw.reshape(b, s, K), pay.reshape(b, s, K))


# speedup vs baseline: 1.5650x; 1.1071x over previous
import functools

import jax
import jax.numpy as jnp
from jax import lax
from jax.experimental import pallas as pl
from jax.experimental.pallas import tpu as pltpu
from jax.experimental.pallas import tpu_sc as plsc

E = 8
K = 2
D = 768
F = 3072
FB = 768
TM = 256
S = 2048
N = S * K
G = N // TM + E
NPAD = G * TM
NW = 32
CHUNK = S // NW
SB = 512


def _router_body(x_ref, cw_ref, cb_ref, w_ref,
                 sel_ref, rw_ref, pay_ref, pos_ref, eid_ref, used_ref):
    x = x_ref[...]
    cw = cw_ref[...]
    logits = jax.lax.dot_general(
        x, cw, (((1,), (1,)), ((), ())), preferred_element_type=jnp.float32)
    logits = logits + cb_ref[...]
    bids = (1.0 / (1.0 + jnp.exp(-logits))) * w_ref[...]
    neg_inf = jnp.float32(float('-inf'))
    ei = jax.lax.broadcasted_iota(jnp.int32, (S, E), 1)
    max1 = jnp.max(bids, axis=1, keepdims=True)
    idx1 = jnp.min(jnp.where(bids == max1, ei, E), axis=1, keepdims=True)
    b2 = jnp.where(ei == idx1, neg_inf, bids)
    max2 = jnp.max(b2, axis=1, keepdims=True)
    idx2 = jnp.min(jnp.where(b2 == max2, ei, E), axis=1, keepdims=True)
    b3 = jnp.where(ei == idx2, neg_inf, b2)
    max3 = jnp.max(b3, axis=1, keepdims=True)
    t = jnp.exp(max2 - max1)
    rw1 = 1.0 / (1.0 + t)
    rw2 = t / (1.0 + t)
    sel_ref[...] = jnp.concatenate([idx1, idx2], axis=1)
    rw_ref[...] = jnp.concatenate([rw1, rw2], axis=1)
    pay_ref[...] = jnp.concatenate([max3, max3], axis=1)

    oh0 = (ei == idx1).astype(jnp.float32)
    oh1 = (ei == idx2).astype(jnp.float32)
    cnt0 = jnp.sum(oh0, axis=0, keepdims=True)
    cnt = cnt0 + jnp.sum(oh1, axis=0, keepdims=True)
    pc = jnp.floor((cnt + (TM - 1)) * (1.0 / TM)) * TM
    ue = jax.lax.broadcasted_iota(jnp.int32, (E, E), 0)
    ve = jax.lax.broadcasted_iota(jnp.int32, (E, E), 1)
    upper = (ue < ve).astype(jnp.float32)
    off = jax.lax.dot(pc, upper, preferred_element_type=jnp.float32)
    incl = off + pc

    ri = jax.lax.broadcasted_iota(jnp.int32, (SB, SB), 0)
    ci = jax.lax.broadcasted_iota(jnp.int32, (SB, SB), 1)
    tri = (ri > ci).astype(jnp.float32)
    prev0 = jnp.zeros((1, E), jnp.float32)
    prev1 = jnp.zeros((1, E), jnp.float32)
    for blk in range(S // SB):
        ohb0 = oh0[blk * SB:(blk + 1) * SB, :]
        ohb1 = oh1[blk * SB:(blk + 1) * SB, :]
        r0 = jax.lax.dot(tri, ohb0, preferred_element_type=jnp.float32) + prev0
        r1 = jax.lax.dot(tri, ohb1, preferred_element_type=jnp.float32) + prev1
        rank0 = jnp.sum(r0 * ohb0, axis=1, keepdims=True)
        rank1 = jnp.sum(r1 * ohb1, axis=1, keepdims=True)
        base0 = jnp.sum(ohb0 * off, axis=1, keepdims=True)
        base1 = jnp.sum(ohb1 * (off + cnt0), axis=1, keepdims=True)
        p0 = (base0 + rank0).astype(jnp.int32)
        p1 = (base1 + rank1).astype(jnp.int32)
        pos_ref[pl.ds(blk * SB, SB), :] = jnp.concatenate([p0, p1], axis=1)
        prev0 = prev0 + jnp.sum(ohb0, axis=0, keepdims=True)
        prev1 = prev1 + jnp.sum(ohb1, axis=0, keepdims=True)

    gi = jax.lax.broadcasted_iota(jnp.int32, (E, G), 1).astype(jnp.float32) * TM
    th = jnp.broadcast_to(incl.reshape(E, 1), (E, G))
    acc = jnp.sum((gi >= th).astype(jnp.int32), axis=0, keepdims=True)
    eid_ref[...] = jnp.minimum(acc, E - 1)
    used_ref[...] = (jnp.sum(pc, axis=1, keepdims=True) *
                     (1.0 / TM)).astype(jnp.int32)


def _scatter_body(x_hbm, p01_hbm, xs_hbm, p0_v, p1_v, rows_v, sem):
    wid = lax.axis_index("s") * 2 + lax.axis_index("c")
    base = wid * CHUNK
    pltpu.sync_copy(p01_hbm.at[0, pl.ds(base, CHUNK)], p0_v)
    pltpu.sync_copy(p01_hbm.at[1, pl.ds(base, CHUNK)], p1_v)
    pltpu.sync_copy(x_hbm.at[pl.ds(base, CHUNK)], rows_v)
    pltpu.async_copy(rows_v, xs_hbm.at[p0_v], sem).wait()
    pltpu.async_copy(rows_v, xs_hbm.at[p1_v], sem).wait()


def _gather_body(ys_hbm, p01_hbm, y0_hbm, y1_hbm, p0_v, p1_v, r0_v, r1_v, sem):
    wid = lax.axis_index("s") * 2 + lax.axis_index("c")
    base = wid * CHUNK
    pltpu.sync_copy(p01_hbm.at[0, pl.ds(base, CHUNK)], p0_v)
    pltpu.sync_copy(p01_hbm.at[1, pl.ds(base, CHUNK)], p1_v)
    pltpu.async_copy(ys_hbm.at[p0_v], r0_v, sem).wait()
    pltpu.async_copy(ys_hbm.at[p1_v], r1_v, sem).wait()
    pltpu.sync_copy(r0_v, y0_hbm.at[pl.ds(base, CHUNK)])
    pltpu.sync_copy(r1_v, y1_hbm.at[pl.ds(base, CHUNK)])


def _combine_body(y0_ref, y1_ref, rw_ref, out_ref):
    w0 = rw_ref[:, 0:1]
    w1 = rw_ref[:, 1:2]
    out_ref[...] = w0 * y0_ref[...] + w1 * y1_ref[...]


def _ffn_body(eid_ref, used_ref, xs_ref, gw_ref, uw_ref, dw_ref, ys_ref,
              gws_ref, uws_ref, dws_ref):
    f = pl.program_id(0)
    g = pl.program_id(1)

    prev_e = eid_ref[jnp.maximum(g, 1) - 1]
    new_block = (g == 0) | (eid_ref[g] != prev_e)

    @pl.when(new_block)
    def _cast():
        gws_ref[...] = gw_ref[0].astype(jnp.bfloat16)
        uws_ref[...] = uw_ref[0].astype(jnp.bfloat16)
        dws_ref[...] = dw_ref[0].astype(jnp.bfloat16)

    @pl.when(g < used_ref[0])
    def _compute():
        xb = xs_ref[pl.ds(g * TM, TM), :].astype(jnp.bfloat16)
        gm = jax.lax.dot(xb, gws_ref[...], preferred_element_type=jnp.float32)
        um = jax.lax.dot(xb, uws_ref[...], preferred_element_type=jnp.float32)
        h = (gm / (1.0 + jnp.exp(-gm))) * um
        contrib = jax.lax.dot(h.astype(jnp.bfloat16), dws_ref[...],
                              preferred_element_type=jnp.float32)

        @pl.when(f == 0)
        def _set():
            ys_ref[pl.ds(g * TM, TM), :] = contrib

        @pl.when(f != 0)
        def _acc():
            ys_ref[pl.ds(g * TM, TM), :] += contrib


def _grouped_ffn(xs, gate_w, up_w, down_w, eid, used):
    grid_spec = pltpu.PrefetchScalarGridSpec(
        num_scalar_prefetch=2,
        grid=(F // FB, G),
        in_specs=[
            pl.BlockSpec((NPAD, D), lambda f, g, eid, used: (0, 0)),
            pl.BlockSpec((1, D, FB), lambda f, g, eid, used: (eid[g], 0, f)),
            pl.BlockSpec((1, D, FB), lambda f, g, eid, used: (eid[g], 0, f)),
            pl.BlockSpec((1, FB, D), lambda f, g, eid, used: (eid[g], f, 0)),
        ],
        out_specs=pl.BlockSpec((NPAD, D), lambda f, g, eid, used: (0, 0)),
        scratch_shapes=[
            pltpu.VMEM((D, FB), jnp.bfloat16),
            pltpu.VMEM((D, FB), jnp.bfloat16),
            pltpu.VMEM((FB, D), jnp.bfloat16),
        ],
    )
    return pl.pallas_call(
        _ffn_body,
        grid_spec=grid_spec,
        out_shape=jax.ShapeDtypeStruct((NPAD, D), jnp.float32),
    )(eid, used, xs, gate_w, up_w, down_w)


@functools.cache
def _sc_mesh():
    return plsc.VectorSubcoreMesh(core_axis_name="c", subcore_axis_name="s")


def _scatter(x, p01):
    kfn = functools.partial(
        pl.kernel, mesh=_sc_mesh(),
        out_type=jax.ShapeDtypeStruct((NPAD, D), jnp.float32),
        scratch_types=[
            pltpu.VMEM((CHUNK,), jnp.int32),
            pltpu.VMEM((CHUNK,), jnp.int32),
            pltpu.VMEM((CHUNK, D), jnp.float32),
            pltpu.SemaphoreType.DMA,
        ],
    )(_scatter_body)
    return kfn(x, p01)


def _gather(ys, p01):
    kfn = functools.partial(
        pl.kernel, mesh=_sc_mesh(),
        out_type=(
            jax.ShapeDtypeStruct((S, D), jnp.float32),
            jax.ShapeDtypeStruct((S, D), jnp.float32),
        ),
        scratch_types=[
            pltpu.VMEM((CHUNK,), jnp.int32),
            pltpu.VMEM((CHUNK,), jnp.int32),
            pltpu.VMEM((CHUNK, D), jnp.float32),
            pltpu.VMEM((CHUNK, D), jnp.float32),
            pltpu.SemaphoreType.DMA,
        ],
    )(_gather_body)
    return kfn(ys, p01)


def kernel(hidden_states, conf_w, conf_b, gate_w, up_w, down_w, wealth):
    b, s, d = hidden_states.shape
    x = hidden_states.reshape(s, d)

    sel, rw, pay, pos, eid2d, used2d = pl.pallas_call(
        _router_body,
        out_shape=(
            jax.ShapeDtypeStruct((S, K), jnp.int32),
            jax.ShapeDtypeStruct((S, K), jnp.float32),
            jax.ShapeDtypeStruct((S, K), jnp.float32),
            jax.ShapeDtypeStruct((S, 2), jnp.int32),
            jax.ShapeDtypeStruct((1, G), jnp.int32),
            jax.ShapeDtypeStruct((1, 1), jnp.int32),
        ),
    )(x, conf_w, conf_b.reshape(1, E), wealth.reshape(1, E))

    p01 = pos.T
    xs = _scatter(x, p01)
    ys = _grouped_ffn(xs, gate_w, up_w, down_w,
                      eid2d.reshape(G), used2d.reshape(1))
    y0, y1 = _gather(ys, p01)

    out = pl.pallas_call(
        _combine_body,
        out_shape=jax.ShapeDtypeStruct((S, D), jnp.float32),
    )(y0, y1, rw)

    return (out.reshape(b, s, d), sel.reshape(b, s, K),
            rw.reshape(b, s, K), pay.reshape(b, s, K))
